# baseline (speedup 1.0000x reference)
import functools

import jax
import jax.numpy as jnp
from jax import lax
from jax.experimental import pallas as pl
from jax.experimental.pallas import tpu as pltpu
from jax.experimental.pallas import tpu_sc as plsc

G = 8
S = 4096
Z = 64
C = 64
H = 4 * Z

NC = 2
NS = 16
L = 16
NW = NC * NS
NPW = (G * S) // NW
WPG = NW // G
BS = 1024

_mesh = plsc.VectorSubcoreMesh(core_axis_name="c", subcore_axis_name="s")
_sc_params = pltpu.CompilerParams(needs_layout_passes=False)


@functools.partial(
    pl.kernel,
    out_type=jax.ShapeDtypeStruct((G, WPG, C * Z), jnp.float32),
    mesh=_mesh,
    scratch_types=[
        pltpu.VMEM((NPW * Z,), jnp.float32),
        pltpu.VMEM((NPW,), jnp.int32),
        pltpu.VMEM((C * Z,), jnp.float32),
    ],
    compiler_params=_sc_params,
)
def _segsum(x_hbm, lab_hbm, sums_hbm, x_v, lab_v, acc_v):
    wid = lax.axis_index("s") * NC + lax.axis_index("c")
    g = wid // WPG
    q = wid % WPG
    base = wid * NPW
    pltpu.sync_copy(x_hbm.at[pl.ds(base * Z, NPW * Z)], x_v)
    pltpu.sync_copy(lab_hbm.at[pl.ds(base, NPW)], lab_v)

    zero = jnp.zeros((L,), jnp.float32)

    def zbody(i, carry):
        acc_v[pl.ds(i * L, L)] = zero
        return carry

    lax.fori_loop(0, (C * Z) // L, zbody, 0)

    iota = lax.iota(jnp.int32, L)

    @plsc.parallel_loop(0, NPW, step=1, unroll=L)
    def _node(node):
        lbl = plsc.load_gather(lab_v, [lax.broadcast(node, (L,))])
        sbase = lbl * Z + iota
        for fc in range(Z // L):
            xv = x_v[pl.ds(node * Z + fc * L, L)]
            plsc.addupdate_scatter(acc_v, [sbase + fc * L], xv)

    pltpu.sync_copy(acc_v, sums_hbm.at[g, q])


def _oc_body(sums_ref, lab_ref, w1_ref, b1_ref, w2_ref, b2_ref, oc_ref):
    s = jnp.sum(sums_ref[0], axis=0)
    lab = lab_ref[0, 0]
    onehot = (lab[:, None] == lax.broadcasted_iota(jnp.int32, (S, C), 1))
    cnt = jnp.sum(onehot.astype(jnp.float32), axis=0)
    m = s / jnp.maximum(cnt, 1.0)[:, None]
    hc = jnp.maximum(
        jnp.dot(m, w1_ref[...], preferred_element_type=jnp.float32) + b1_ref[0],
        0.0,
    )
    oc_ref[0] = jnp.dot(hc, w2_ref[...],
                        preferred_element_type=jnp.float32) + b2_ref[0]


_oc_kernel = pl.pallas_call(
    _oc_body,
    grid=(G,),
    in_specs=[
        pl.BlockSpec((1, WPG, C, Z), lambda g: (g, 0, 0, 0)),
        pl.BlockSpec((1, 1, S), lambda g: (g, 0, 0)),
        pl.BlockSpec((Z, H), lambda g: (0, 0)),
        pl.BlockSpec((1, H), lambda g: (0, 0)),
        pl.BlockSpec((H, Z), lambda g: (0, 0)),
        pl.BlockSpec((1, Z), lambda g: (0, 0)),
    ],
    out_specs=pl.BlockSpec((1, C, Z), lambda g: (g, 0, 0)),
    out_shape=jax.ShapeDtypeStruct((G, C, Z), jnp.float32),
)


def _resid_body(x_ref, lab_ref, oc_ref, out_ref):
    lab = lab_ref[0, 0, 0]
    onehot = (lab[:, None] == lax.broadcasted_iota(jnp.int32, (BS, C), 1))
    out_ref[0] = x_ref[0] + jnp.dot(
        onehot.astype(jnp.float32), oc_ref[0],
        preferred_element_type=jnp.float32,
    )


_resid_kernel = pl.pallas_call(
    _resid_body,
    grid=(G, S // BS),
    in_specs=[
        pl.BlockSpec((1, BS, Z), lambda g, j: (g, j, 0)),
        pl.BlockSpec((1, 1, 1, BS), lambda g, j: (g, j, 0, 0)),
        pl.BlockSpec((1, C, Z), lambda g, j: (g, 0, 0)),
    ],
    out_specs=pl.BlockSpec((1, BS, Z), lambda g, j: (g, j, 0)),
    out_shape=jax.ShapeDtypeStruct((G, S, Z), jnp.float32),
)


def kernel(graph_input_raw, graph_label, W1, b1, W2, b2):
    x_flat = graph_input_raw.reshape(-1)
    lab_flat = graph_label.reshape(-1).astype(jnp.int32)
    sums = _segsum(x_flat, lab_flat)
    oc = _oc_kernel(
        sums.reshape(G, WPG, C, Z),
        graph_label.reshape(G, 1, S),
        W1,
        b1.reshape(1, H),
        W2,
        b2.reshape(1, Z),
    )
    return _resid_kernel(
        graph_input_raw,
        graph_label.astype(jnp.int32).reshape(G, S // BS, 1, BS),
        oc,
    )


# speedup vs baseline: 1.2266x; 1.2266x over previous
import functools

import jax
import jax.numpy as jnp
from jax import lax
from jax.experimental import pallas as pl
from jax.experimental.pallas import tpu as pltpu
from jax.experimental.pallas import tpu_sc as plsc

G = 8
S = 4096
Z = 64
C = 64
H = 4 * Z

NC = 2
NS = 16
L = 16
NW = NC * NS
NPW = (G * S) // NW
NWG = NW // G
GROUPS = NPW // L

_mesh = plsc.VectorSubcoreMesh(core_axis_name="c", subcore_axis_name="s")
_sc_params = pltpu.CompilerParams(needs_layout_passes=False)


@functools.partial(
    pl.kernel,
    out_type=jax.ShapeDtypeStruct((G, NWG, C * Z), jnp.float32),
    mesh=_mesh,
    scratch_types=[
        pltpu.VMEM((NPW * Z,), jnp.float32),
        pltpu.VMEM((NPW,), jnp.int32),
        pltpu.VMEM((C * Z,), jnp.float32),
    ],
    compiler_params=_sc_params,
)
def _segsum(x_hbm, lab_hbm, sums_hbm, x_v, lab_v, acc_v):
    wid = lax.axis_index("s") * NC + lax.axis_index("c")
    g = wid // NWG
    q = wid % NWG
    base = wid * NPW
    pltpu.sync_copy(x_hbm.at[pl.ds(base * Z, NPW * Z)], x_v)
    pltpu.sync_copy(lab_hbm.at[pl.ds(base, NPW)], lab_v)

    zero = jnp.zeros((L,), jnp.float32)

    def zbody(i, carry):
        acc_v[pl.ds(i * L, L)] = zero
        return carry

    lax.fori_loop(0, (C * Z) // L, zbody, 0)

    iota = lax.iota(jnp.int32, L)

    @plsc.parallel_loop(0, NPW, step=1, unroll=L)
    def _node(node):
        lbl = plsc.load_gather(lab_v, [lax.broadcast(node, (L,))])
        sbase = lbl * Z + iota
        for fc in range(Z // L):
            xv = x_v[pl.ds(node * Z + fc * L, L)]
            plsc.addupdate_scatter(acc_v, [sbase + fc * L], xv)

    pltpu.sync_copy(acc_v, sums_hbm.at[g, q])


def _tc_tail_body(sums_ref, lab_ref, w1_ref, b1_ref, w2_ref, b2_ref, x_ref,
                  out_ref):
    s = jnp.sum(sums_ref[0], axis=0)
    lab = lab_ref[0, 0]
    onehot = (lab[:, None] == lax.broadcasted_iota(jnp.int32, (S, C), 1))
    onehot = onehot.astype(jnp.float32)
    cnt = jnp.sum(onehot, axis=0)
    m = s / jnp.maximum(cnt, 1.0)[:, None]
    hc = jnp.maximum(
        jnp.dot(m, w1_ref[...], preferred_element_type=jnp.float32) + b1_ref[0],
        0.0,
    )
    oc = jnp.dot(hc, w2_ref[...], preferred_element_type=jnp.float32) + b2_ref[0]
    out_ref[0] = x_ref[0] + jnp.dot(
        onehot, oc, preferred_element_type=jnp.float32
    )


_tc_tail = pl.pallas_call(
    _tc_tail_body,
    grid=(G,),
    in_specs=[
        pl.BlockSpec((1, NWG, C, Z), lambda g: (g, 0, 0, 0)),
        pl.BlockSpec((1, 1, S), lambda g: (g, 0, 0)),
        pl.BlockSpec((Z, H), lambda g: (0, 0)),
        pl.BlockSpec((1, H), lambda g: (0, 0)),
        pl.BlockSpec((H, Z), lambda g: (0, 0)),
        pl.BlockSpec((1, Z), lambda g: (0, 0)),
        pl.BlockSpec((1, S, Z), lambda g: (g, 0, 0)),
    ],
    out_specs=pl.BlockSpec((1, S, Z), lambda g: (g, 0, 0)),
    out_shape=jax.ShapeDtypeStruct((G, S, Z), jnp.float32),
)


@functools.partial(
    pl.kernel,
    out_type=jax.ShapeDtypeStruct((G * S * Z,), jnp.float32),
    mesh=_mesh,
    scratch_types=[
        pltpu.VMEM((NPW * Z,), jnp.float32),
        pltpu.VMEM((NPW,), jnp.int32),
        pltpu.VMEM((C * Z,), jnp.float32),
    ],
    compiler_params=_sc_params,
)
def _gather_add(x_hbm, lab_hbm, oc_hbm, out_hbm, x_v, lab_v, oc_v):
    wid = lax.axis_index("s") * NC + lax.axis_index("c")
    g = wid // NWG
    base = wid * NPW
    pltpu.sync_copy(x_hbm.at[pl.ds(base * Z, NPW * Z)], x_v)
    pltpu.sync_copy(lab_hbm.at[pl.ds(base, NPW)], lab_v)
    pltpu.sync_copy(oc_hbm.at[pl.ds(g * (C * Z), C * Z)], oc_v)

    iota = lax.iota(jnp.int32, L)

    @plsc.parallel_loop(0, NPW, step=1, unroll=L)
    def _node(node):
        lbl = plsc.load_gather(lab_v, [lax.broadcast(node, (L,))])
        obase = lbl * Z + iota
        for fc in range(Z // L):
            val = plsc.load_gather(oc_v, [obase + fc * L])
            plsc.addupdate(x_v.at[pl.ds(node * Z + fc * L, L)], val)

    pltpu.sync_copy(x_v, out_hbm.at[pl.ds(base * Z, NPW * Z)])


def kernel(graph_input_raw, graph_label, W1, b1, W2, b2):
    x_flat = graph_input_raw.reshape(-1)
    lab_flat = graph_label.reshape(-1)
    sums = _segsum(x_flat, lab_flat)
    return _tc_tail(
        sums.reshape(G, NWG, C, Z),
        graph_label.reshape(G, 1, S),
        W1,
        b1.reshape(1, H),
        W2,
        b2.reshape(1, Z),
        graph_input_raw,
    )


# speedup vs baseline: 1.2269x; 1.0002x over previous
import functools

import jax
import jax.numpy as jnp
from jax import lax
from jax.experimental import pallas as pl
from jax.experimental.pallas import tpu as pltpu
from jax.experimental.pallas import tpu_sc as plsc

G = 8
S = 4096
Z = 64
C = 64
H = 4 * Z

NC = 2
NS = 16
L = 16
NW = NC * NS
NPW = (G * S) // NW
NWG = NW // G
GROUPS = NPW // L

_mesh = plsc.VectorSubcoreMesh(core_axis_name="c", subcore_axis_name="s")
_sc_params = pltpu.CompilerParams(needs_layout_passes=False)


@functools.partial(
    pl.kernel,
    out_type=jax.ShapeDtypeStruct((G, NWG, C * Z), jnp.float32),
    mesh=_mesh,
    scratch_types=[
        pltpu.VMEM((NPW * Z,), jnp.float32),
        pltpu.VMEM((NPW,), jnp.int32),
        pltpu.VMEM((C * Z,), jnp.float32),
    ],
    compiler_params=_sc_params,
)
def _segsum(x_hbm, lab_hbm, sums_hbm, x_v, lab_v, acc_v):
    wid = lax.axis_index("s") * NC + lax.axis_index("c")
    g = wid // NWG
    q = wid % NWG
    base = wid * NPW
    pltpu.sync_copy(x_hbm.at[pl.ds(base * Z, NPW * Z)], x_v)
    pltpu.sync_copy(lab_hbm.at[pl.ds(base, NPW)], lab_v)

    zero = jnp.zeros((L,), jnp.float32)

    def zbody(i, carry):
        acc_v[pl.ds(i * L, L)] = zero
        return carry

    lax.fori_loop(0, (C * Z) // L, zbody, 0)

    iota = lax.iota(jnp.int32, L)

    @plsc.parallel_loop(0, NPW, step=1, unroll=L)
    def _node(node):
        lbl = plsc.load_gather(lab_v, [lax.broadcast(node, (L,))])
        sbase = lbl * Z + iota
        for fc in range(Z // L):
            xv = x_v[pl.ds(node * Z + fc * L, L)]
            plsc.addupdate_scatter(acc_v, [sbase + fc * L], xv)

    pltpu.sync_copy(acc_v, sums_hbm.at[g, q])


def _tc_tail_body(sums_ref, lab_ref, w1_ref, b1_ref, w2_ref, b2_ref, x_ref,
                  out_ref):
    s = jnp.sum(sums_ref[0], axis=0)
    lab = lab_ref[0, 0]
    onehot = (lab[:, None] == lax.broadcasted_iota(jnp.int32, (S, C), 1))
    onehot = onehot.astype(jnp.float32)
    cnt = jnp.sum(onehot, axis=0)
    m = s / jnp.maximum(cnt, 1.0)[:, None]
    hc = jnp.maximum(
        jnp.dot(m, w1_ref[...], preferred_element_type=jnp.float32) + b1_ref[0],
        0.0,
    )
    oc = jnp.dot(hc, w2_ref[...], preferred_element_type=jnp.float32) + b2_ref[0]
    out_ref[0] = x_ref[0] + jnp.dot(
        onehot, oc, preferred_element_type=jnp.float32
    )


_tc_tail = pl.pallas_call(
    _tc_tail_body,
    grid=(G,),
    in_specs=[
        pl.BlockSpec((1, NWG, C, Z), lambda g: (g, 0, 0, 0)),
        pl.BlockSpec((1, 1, S), lambda g: (g, 0, 0)),
        pl.BlockSpec((Z, H), lambda g: (0, 0)),
        pl.BlockSpec((1, H), lambda g: (0, 0)),
        pl.BlockSpec((H, Z), lambda g: (0, 0)),
        pl.BlockSpec((1, Z), lambda g: (0, 0)),
        pl.BlockSpec((1, S, Z), lambda g: (g, 0, 0)),
    ],
    out_specs=pl.BlockSpec((1, S, Z), lambda g: (g, 0, 0)),
    out_shape=jax.ShapeDtypeStruct((G, S, Z), jnp.float32),
    compiler_params=pltpu.CompilerParams(
        dimension_semantics=("parallel",),
    ),
)


def kernel(graph_input_raw, graph_label, W1, b1, W2, b2):
    x_flat = graph_input_raw.reshape(-1)
    lab_flat = graph_label.reshape(-1)
    sums = _segsum(x_flat, lab_flat)
    return _tc_tail(
        sums.reshape(G, NWG, C, Z),
        graph_label.reshape(G, 1, S),
        W1,
        b1.reshape(1, H),
        W2,
        b2.reshape(1, Z),
        graph_input_raw,
    )


# speedup vs baseline: 1.2595x; 1.0266x over previous
import functools

import jax
import jax.numpy as jnp
from jax import lax
from jax.experimental import pallas as pl
from jax.experimental.pallas import tpu as pltpu
from jax.experimental.pallas import tpu_sc as plsc

G = 8
S = 4096
Z = 64
C = 64
H = 4 * Z

NC = 2
NS = 16
L = 16
NW = NC * NS
NPW = (G * S) // NW
NWG = NW // G
GROUPS = NPW // L

_mesh = plsc.VectorSubcoreMesh(core_axis_name="c", subcore_axis_name="s")
_sc_params = pltpu.CompilerParams(needs_layout_passes=False)


@functools.partial(
    pl.kernel,
    out_type=jax.ShapeDtypeStruct((G, NWG, C * Z), jnp.float32),
    mesh=_mesh,
    scratch_types=[
        pltpu.VMEM((NPW * Z,), jnp.float32),
        pltpu.VMEM((NPW,), jnp.int32),
        pltpu.VMEM((C * Z,), jnp.float32),
    ],
    compiler_params=_sc_params,
)
def _segsum(x_hbm, lab_hbm, sums_hbm, x_v, lab_v, acc_v):
    wid = lax.axis_index("s") * NC + lax.axis_index("c")
    g = wid // NWG
    q = wid % NWG
    base = wid * NPW
    pltpu.sync_copy(x_hbm.at[pl.ds(base * Z, NPW * Z)], x_v)
    pltpu.sync_copy(lab_hbm.at[pl.ds(base, NPW)], lab_v)

    zero = jnp.zeros((L,), jnp.float32)

    def zbody(i, carry):
        acc_v[pl.ds(i * L, L)] = zero
        return carry

    lax.fori_loop(0, (C * Z) // L, zbody, 0)

    iota = lax.iota(jnp.int32, L)

    @plsc.parallel_loop(0, NPW, step=1, unroll=L)
    def _node(node):
        lbl = plsc.load_gather(lab_v, [lax.broadcast(node, (L,))])
        sbase = lbl * Z + iota
        for fc in range(Z // L):
            xv = x_v[pl.ds(node * Z + fc * L, L)]
            plsc.addupdate_scatter(acc_v, [sbase + fc * L], xv)

    pltpu.sync_copy(acc_v, sums_hbm.at[g, q])


G2 = 2


def _tc_tail_body(sums_ref, lab_ref, w1_ref, b1_ref, w2_ref, b2_ref, x_ref,
                  out_ref):
    for i in range(G2):
        s = jnp.sum(sums_ref[i], axis=0)
        lab = lab_ref[i, 0]
        onehot = (lab[:, None] == lax.broadcasted_iota(jnp.int32, (S, C), 1))
        onehot = onehot.astype(jnp.float32)
        cnt = jnp.sum(onehot, axis=0)
        m = s / jnp.maximum(cnt, 1.0)[:, None]
        hc = jnp.maximum(
            jnp.dot(m, w1_ref[...], preferred_element_type=jnp.float32)
            + b1_ref[0],
            0.0,
        )
        oc = jnp.dot(hc, w2_ref[...],
                     preferred_element_type=jnp.float32) + b2_ref[0]
        out_ref[i] = x_ref[i] + jnp.dot(
            onehot, oc, preferred_element_type=jnp.float32
        )


_tc_tail = pl.pallas_call(
    _tc_tail_body,
    grid=(G // G2,),
    in_specs=[
        pl.BlockSpec((G2, NWG, C, Z), lambda g: (g, 0, 0, 0)),
        pl.BlockSpec((G2, 1, S), lambda g: (g, 0, 0)),
        pl.BlockSpec((Z, H), lambda g: (0, 0)),
        pl.BlockSpec((1, H), lambda g: (0, 0)),
        pl.BlockSpec((H, Z), lambda g: (0, 0)),
        pl.BlockSpec((1, Z), lambda g: (0, 0)),
        pl.BlockSpec((G2, S, Z), lambda g: (g, 0, 0)),
    ],
    out_specs=pl.BlockSpec((G2, S, Z), lambda g: (g, 0, 0)),
    out_shape=jax.ShapeDtypeStruct((G, S, Z), jnp.float32),
    compiler_params=pltpu.CompilerParams(
        dimension_semantics=("parallel",),
    ),
)


def kernel(graph_input_raw, graph_label, W1, b1, W2, b2):
    x_flat = graph_input_raw.reshape(-1)
    lab_flat = graph_label.reshape(-1)
    sums = _segsum(x_flat, lab_flat)
    return _tc_tail(
        sums.reshape(G, NWG, C, Z),
        graph_label.reshape(G, 1, S),
        W1,
        b1.reshape(1, H),
        W2,
        b2.reshape(1, Z),
        graph_input_raw,
    )


# speedup vs baseline: 1.2741x; 1.0115x over previous
import functools

import jax
import jax.numpy as jnp
from jax import lax
from jax.experimental import pallas as pl
from jax.experimental.pallas import tpu as pltpu
from jax.experimental.pallas import tpu_sc as plsc

G = 8
S = 4096
Z = 64
C = 64
H = 4 * Z

NC = 2
NS = 16
L = 16
NW = NC * NS
NPW = (G * S) // NW
NWG = NW // G
GROUPS = NPW // L

_mesh = plsc.VectorSubcoreMesh(core_axis_name="c", subcore_axis_name="s")
_sc_params = pltpu.CompilerParams(needs_layout_passes=False)


@functools.partial(
    pl.kernel,
    out_type=jax.ShapeDtypeStruct((G, NWG, C * Z), jnp.float32),
    mesh=_mesh,
    scratch_types=[
        pltpu.VMEM((NPW * Z,), jnp.float32),
        pltpu.VMEM((NPW,), jnp.int32),
        pltpu.VMEM((C * Z,), jnp.float32),
    ],
    compiler_params=_sc_params,
)
def _segsum(x_hbm, lab_hbm, sums_hbm, x_v, lab_v, acc_v):
    wid = lax.axis_index("s") * NC + lax.axis_index("c")
    g = wid // NWG
    q = wid % NWG
    base = wid * NPW
    pltpu.sync_copy(x_hbm.at[pl.ds(base * Z, NPW * Z)], x_v)
    pltpu.sync_copy(lab_hbm.at[pl.ds(base, NPW)], lab_v)

    zero = jnp.zeros((L,), jnp.float32)

    def zbody(i, carry):
        acc_v[pl.ds(i * L, L)] = zero
        return carry

    lax.fori_loop(0, (C * Z) // L, zbody, 0)

    iota = lax.iota(jnp.int32, L)

    @plsc.parallel_loop(0, NPW, step=1, unroll=L)
    def _node(node):
        lbl = plsc.load_gather(lab_v, [lax.broadcast(node, (L,))])
        sbase = lbl * Z + iota
        for fc in range(Z // L):
            xv = x_v[pl.ds(node * Z + fc * L, L)]
            plsc.addupdate_scatter(acc_v, [sbase + fc * L], xv)

    pltpu.sync_copy(acc_v, sums_hbm.at[g, q])


G2 = 4


def _tc_tail_body(sums_ref, lab_ref, w1_ref, b1_ref, w2_ref, b2_ref, x_ref,
                  out_ref):
    for i in range(G2):
        s = jnp.sum(sums_ref[i], axis=0)
        lab = lab_ref[i, 0]
        onehot = (lab[:, None] == lax.broadcasted_iota(jnp.int32, (S, C), 1))
        onehot = onehot.astype(jnp.float32)
        cnt = jnp.sum(onehot, axis=0)
        m = s / jnp.maximum(cnt, 1.0)[:, None]
        hc = jnp.maximum(
            jnp.dot(m, w1_ref[...], preferred_element_type=jnp.float32)
            + b1_ref[0],
            0.0,
        )
        oc = jnp.dot(hc, w2_ref[...],
                     preferred_element_type=jnp.float32) + b2_ref[0]
        out_ref[i] = x_ref[i] + jnp.dot(
            onehot, oc, preferred_element_type=jnp.float32
        )


_tc_tail = pl.pallas_call(
    _tc_tail_body,
    grid=(G // G2,),
    in_specs=[
        pl.BlockSpec((G2, NWG, C, Z), lambda g: (g, 0, 0, 0)),
        pl.BlockSpec((G2, 1, S), lambda g: (g, 0, 0)),
        pl.BlockSpec((Z, H), lambda g: (0, 0)),
        pl.BlockSpec((1, H), lambda g: (0, 0)),
        pl.BlockSpec((H, Z), lambda g: (0, 0)),
        pl.BlockSpec((1, Z), lambda g: (0, 0)),
        pl.BlockSpec((G2, S, Z), lambda g: (g, 0, 0)),
    ],
    out_specs=pl.BlockSpec((G2, S, Z), lambda g: (g, 0, 0)),
    out_shape=jax.ShapeDtypeStruct((G, S, Z), jnp.float32),
    compiler_params=pltpu.CompilerParams(
        dimension_semantics=("parallel",),
    ),
)


def kernel(graph_input_raw, graph_label, W1, b1, W2, b2):
    x_flat = graph_input_raw.reshape(-1)
    lab_flat = graph_label.reshape(-1)
    sums = _segsum(x_flat, lab_flat)
    return _tc_tail(
        sums.reshape(G, NWG, C, Z),
        graph_label.reshape(G, 1, S),
        W1,
        b1.reshape(1, H),
        W2,
        b2.reshape(1, Z),
        graph_input_raw,
    )
